# baseline (speedup 1.0000x reference)
import functools

import jax
import jax.numpy as jnp
from jax import lax
from jax.experimental import pallas as pl
from jax.experimental.pallas import tpu as pltpu
from jax.experimental.pallas import tpu_sc as plsc

EMBEDDING_DIM = 32
NUM_ROWS = 1000000
BATCH, SEQ = 16384, 26
B_TOTAL = BATCH * SEQ
NUM_CORES, NUM_SUBCORES = 2, 16
NW = NUM_CORES * NUM_SUBCORES

NBLK = NUM_ROWS // 128
MAIN_ROWS = NBLK * 128
J_FULL = NBLK // NW
TAIL_W = NBLK - J_FULL * NW

B_PER_W = B_TOTAL // NW
CHUNK = 1664
NCHUNK = B_PER_W // CHUNK
NBUF = 2

_mesh = plsc.VectorSubcoreMesh(core_axis_name="c", subcore_axis_name="s")


def _worker_id():
    return lax.axis_index("s") * NUM_CORES + lax.axis_index("c")


def _transpose_block(blk, tbuf, width):
    iota = lax.iota(jnp.int32, 16)
    zeros = jnp.zeros((16,), jnp.int32)

    def body(eg, _):
        for u in range(8):
            e = eg * 8 + u
            cols = zeros + e
            for h in range(2):
                v = plsc.load_gather(blk, [iota + 16 * h, cols])
                tbuf[pl.ds(e * 32 + 16 * h, 16)] = v
        return 0

    lax.fori_loop(0, width // 8, body, 0)


@functools.partial(
    pl.kernel,
    out_type=jax.ShapeDtypeStruct((NUM_ROWS * EMBEDDING_DIM,), jnp.float32),
    mesh=_mesh,
    scratch_types=(
        [pltpu.VMEM((32, 128), jnp.float32) for _ in range(2)]
        + [pltpu.VMEM((4096,), jnp.float32) for _ in range(2)]
        + [pltpu.SemaphoreType.DMA for _ in range(4)]
    ),
    compiler_params=pltpu.CompilerParams(use_tc_tiling_on_sc=True,
                                         needs_layout_passes=False),
)
def _sc_repack(wt, tailp, out,
               blk0, blk1, tb0, tb1, si0, si1, so0, so1):
    blks, tbs = (blk0, blk1), (tb0, tb1)
    sis, sos = (si0, si1), (so0, so1)
    w = _worker_id()

    def c_of(j):
        return j * NW + w

    def start_in(j, slot):
        pltpu.async_copy(
            wt.at[:, pl.ds(c_of(j) * 128, 128)], blks[slot], sis[slot])

    def wait_in(slot):
        pltpu.make_async_copy(
            wt.at[:, pl.ds(0, 128)], blks[slot], sis[slot]).wait()

    def start_out(j, slot):
        pltpu.async_copy(
            tbs[slot], out.at[pl.ds(c_of(j) * 4096, 4096)], sos[slot])

    def wait_out(slot):
        pltpu.make_async_copy(
            tbs[slot], out.at[pl.ds(0, 4096)], sos[slot]).wait()

    start_in(0, 0)
    start_in(1, 1)

    def body(jj, _):
        for b in range(2):
            j = jj * 2 + b
            wait_in(b)

            @pl.when(jj > 0)
            def _():
                wait_out(b)

            _transpose_block(blks[b], tbs[b], 128)

            @pl.when(j + 2 < J_FULL)
            def _():
                start_in(j + 2, b)

            start_out(j, b)
        return 0

    lax.fori_loop(0, J_FULL // 2, body, 0)
    wait_out(0)
    wait_out(1)

    @pl.when(w < TAIL_W)
    def _():
        start_in(J_FULL, 0)
        wait_in(0)
        _transpose_block(blk0, tb0, 128)
        start_out(J_FULL, 0)
        wait_out(0)

    @pl.when(w == TAIL_W)
    def _():
        pltpu.async_copy(tailp, blk1, si1)
        pltpu.make_async_copy(tailp, blk1, si1).wait()
        _transpose_block(blk1, tb1, 64)
        pltpu.async_copy(
            tb1.at[pl.ds(0, (NUM_ROWS - MAIN_ROWS) * EMBEDDING_DIM)],
            out.at[pl.ds(MAIN_ROWS * EMBEDDING_DIM,
                         (NUM_ROWS - MAIN_ROWS) * EMBEDDING_DIM)],
            so1)
        pltpu.make_async_copy(
            tb1.at[pl.ds(0, (NUM_ROWS - MAIN_ROWS) * EMBEDDING_DIM)],
            out.at[pl.ds(MAIN_ROWS * EMBEDDING_DIM,
                         (NUM_ROWS - MAIN_ROWS) * EMBEDDING_DIM)],
            so1).wait()


@functools.partial(
    pl.kernel,
    out_type=jax.ShapeDtypeStruct((B_TOTAL, EMBEDDING_DIM), jnp.float32),
    mesh=_mesh,
    scratch_types=(
        [pltpu.VMEM((CHUNK,), jnp.int32) for _ in range(NBUF)]
        + [pltpu.VMEM((CHUNK, EMBEDDING_DIM), jnp.float32) for _ in range(NBUF)]
        + [pltpu.SemaphoreType.DMA for _ in range(NBUF)]
    ),
    compiler_params=pltpu.CompilerParams(use_tc_tiling_on_sc=False),
)
def _sc_gather(idx_hbm, table_hbm, out_hbm,
               idx0, idx1, rows0, rows1, sem0, sem1):
    idx_bufs = (idx0, idx1)
    row_bufs = (rows0, rows1)
    sems = (sem0, sem1)
    base = _worker_id() * B_PER_W

    def start(g, slot):
        off = base + g * CHUNK
        pltpu.sync_copy(idx_hbm.at[pl.ds(off, CHUNK)], idx_bufs[slot])
        return pltpu.async_copy(table_hbm.at[idx_bufs[slot]], row_bufs[slot],
                                sems[slot])

    inflight = [None] * NBUF
    inflight[0] = start(0, 0)
    for g in range(NCHUNK):
        slot = g % NBUF
        nxt = (g + 1) % NBUF
        if g + 1 < NCHUNK:
            inflight[nxt] = start(g + 1, nxt)
        inflight[slot].wait()
        pltpu.sync_copy(row_bufs[slot],
                        out_hbm.at[pl.ds(base + g * CHUNK, CHUNK)])


def kernel(input_ids, weight):
    flat = input_ids.reshape(-1).astype(jnp.int32)
    wt = weight.T
    tailp = jnp.pad(wt[:, MAIN_ROWS:], ((0, 0), (0, 128 - (NUM_ROWS - MAIN_ROWS))))
    table = _sc_repack(wt, tailp).reshape(NUM_ROWS, EMBEDDING_DIM)
    out = _sc_gather(flat, table)
    return out.reshape(*input_ids.shape, EMBEDDING_DIM)


# speedup vs baseline: 1.4423x; 1.4423x over previous
import functools

import jax
import jax.numpy as jnp
from jax import lax
from jax.experimental import pallas as pl
from jax.experimental.pallas import tpu as pltpu
from jax.experimental.pallas import tpu_sc as plsc

EMBEDDING_DIM = 32
NUM_ROWS = 1000000
BATCH, SEQ = 16384, 26
B_TOTAL = BATCH * SEQ
NUM_CORES, NUM_SUBCORES = 2, 16
NW = NUM_CORES * NUM_SUBCORES

NBLK = NUM_ROWS // 128
MAIN_ROWS = NBLK * 128
J_FULL = NBLK // NW
TAIL_W = NBLK - J_FULL * NW

B_PER_W = B_TOTAL // NW
CHUNK = 1664
NCHUNK = B_PER_W // CHUNK
NBUF = 2

_mesh = plsc.VectorSubcoreMesh(core_axis_name="c", subcore_axis_name="s")


def _worker_id():
    return lax.axis_index("s") * NUM_CORES + lax.axis_index("c")


def _transpose_block(blk, tbuf, width):
    iota = lax.iota(jnp.int32, 16)
    zeros = jnp.zeros((16,), jnp.int32)
    ng = width // 16
    cols = [iota + 16 * g for g in range(ng)]
    sidx = [(iota + 16 * g) * 32 for g in range(ng)]

    @plsc.parallel_loop(0, 32, unroll=8)
    def _(d):
        dvec = zeros + d
        for g in range(ng):
            v = plsc.load_gather(blk, [dvec, cols[g]])
            plsc.store_scatter(tbuf, [sidx[g] + d], v)


@functools.partial(
    pl.kernel,
    out_type=jax.ShapeDtypeStruct((NUM_ROWS * EMBEDDING_DIM,), jnp.float32),
    mesh=_mesh,
    scratch_types=(
        [pltpu.VMEM((32, 128), jnp.float32) for _ in range(2)]
        + [pltpu.VMEM((4096,), jnp.float32) for _ in range(2)]
        + [pltpu.SemaphoreType.DMA for _ in range(4)]
    ),
    compiler_params=pltpu.CompilerParams(use_tc_tiling_on_sc=True,
                                         needs_layout_passes=False),
)
def _sc_repack(wt, tailp, out,
               blk0, blk1, tb0, tb1, si0, si1, so0, so1):
    blks, tbs = (blk0, blk1), (tb0, tb1)
    sis, sos = (si0, si1), (so0, so1)
    w = _worker_id()

    def c_of(j):
        return j * NW + w

    def start_in(j, slot):
        pltpu.async_copy(
            wt.at[:, pl.ds(c_of(j) * 128, 128)], blks[slot], sis[slot])

    def wait_in(slot):
        pltpu.make_async_copy(
            wt.at[:, pl.ds(0, 128)], blks[slot], sis[slot]).wait()

    def start_out(j, slot):
        pltpu.async_copy(
            tbs[slot], out.at[pl.ds(c_of(j) * 4096, 4096)], sos[slot])

    def wait_out(slot):
        pltpu.make_async_copy(
            tbs[slot], out.at[pl.ds(0, 4096)], sos[slot]).wait()

    start_in(0, 0)
    start_in(1, 1)

    def body(jj, _):
        for b in range(2):
            j = jj * 2 + b
            wait_in(b)

            @pl.when(jj > 0)
            def _():
                wait_out(b)

            _transpose_block(blks[b], tbs[b], 128)

            @pl.when(j + 2 < J_FULL)
            def _():
                start_in(j + 2, b)

            start_out(j, b)
        return 0

    lax.fori_loop(0, J_FULL // 2, body, 0)
    wait_out(0)
    wait_out(1)

    @pl.when(w < TAIL_W)
    def _():
        start_in(J_FULL, 0)
        wait_in(0)
        _transpose_block(blk0, tb0, 128)
        start_out(J_FULL, 0)
        wait_out(0)

    @pl.when(w == TAIL_W)
    def _():
        pltpu.async_copy(tailp, blk1, si1)
        pltpu.make_async_copy(tailp, blk1, si1).wait()
        _transpose_block(blk1, tb1, 64)
        pltpu.async_copy(
            tb1.at[pl.ds(0, (NUM_ROWS - MAIN_ROWS) * EMBEDDING_DIM)],
            out.at[pl.ds(MAIN_ROWS * EMBEDDING_DIM,
                         (NUM_ROWS - MAIN_ROWS) * EMBEDDING_DIM)],
            so1)
        pltpu.make_async_copy(
            tb1.at[pl.ds(0, (NUM_ROWS - MAIN_ROWS) * EMBEDDING_DIM)],
            out.at[pl.ds(MAIN_ROWS * EMBEDDING_DIM,
                         (NUM_ROWS - MAIN_ROWS) * EMBEDDING_DIM)],
            so1).wait()


@functools.partial(
    pl.kernel,
    out_type=jax.ShapeDtypeStruct((B_TOTAL, EMBEDDING_DIM), jnp.float32),
    mesh=_mesh,
    scratch_types=(
        [pltpu.VMEM((CHUNK,), jnp.int32) for _ in range(NBUF)]
        + [pltpu.VMEM((CHUNK, EMBEDDING_DIM), jnp.float32) for _ in range(NBUF)]
        + [pltpu.SemaphoreType.DMA for _ in range(NBUF)]
    ),
    compiler_params=pltpu.CompilerParams(use_tc_tiling_on_sc=False),
)
def _sc_gather(idx_hbm, table_hbm, out_hbm,
               idx0, idx1, rows0, rows1, sem0, sem1):
    idx_bufs = (idx0, idx1)
    row_bufs = (rows0, rows1)
    sems = (sem0, sem1)
    base = _worker_id() * B_PER_W

    def start(g, slot):
        off = base + g * CHUNK
        pltpu.sync_copy(idx_hbm.at[pl.ds(off, CHUNK)], idx_bufs[slot])
        return pltpu.async_copy(table_hbm.at[idx_bufs[slot]], row_bufs[slot],
                                sems[slot])

    inflight = [None] * NBUF
    inflight[0] = start(0, 0)
    for g in range(NCHUNK):
        slot = g % NBUF
        nxt = (g + 1) % NBUF
        if g + 1 < NCHUNK:
            inflight[nxt] = start(g + 1, nxt)
        inflight[slot].wait()
        pltpu.sync_copy(row_bufs[slot],
                        out_hbm.at[pl.ds(base + g * CHUNK, CHUNK)])


def kernel(input_ids, weight):
    flat = input_ids.reshape(-1).astype(jnp.int32)
    wt = weight.T
    tailp = jnp.pad(wt[:, MAIN_ROWS:], ((0, 0), (0, 128 - (NUM_ROWS - MAIN_ROWS))))
    table = _sc_repack(wt, tailp).reshape(NUM_ROWS, EMBEDDING_DIM)
    out = _sc_gather(flat, table)
    return out.reshape(*input_ids.shape, EMBEDDING_DIM)
